# baseline (speedup 1.0000x reference)
import functools

import jax
import jax.numpy as jnp
from jax import lax
from jax.experimental import pallas as pl
from jax.experimental.pallas import tpu as pltpu
from jax.experimental.pallas import tpu_sc as plsc

N = 8000
D = 128
E = 128000
NPER = 1000
NSEQ = 999
NG = 8

NCORES = 2
NSUB = 16
H = D // 2
EPT = E // NSUB
CHUNK = 80
NCHUNK = EPT // CHUNK
ROWS_PT = N // NSUB


def _leaky(v):
    return jnp.maximum(v, 0.2 * v)


RB = 1000
NBLK = N // RB


def _project(t, wl_ref, bl_ref, wa1_ref, wa2_ref, ba_ref,
             ai_ref, g_ref, ps_ref):
    xl = jnp.dot(t, wl_ref[...].T, preferred_element_type=jnp.float32) + bl_ref[...]
    ai = jnp.dot(xl, wa1_ref[...].T, preferred_element_type=jnp.float32) + ba_ref[...]
    aj = jnp.dot(xl, wa2_ref[...].T, preferred_element_type=jnp.float32)
    ps = jnp.exp(_leaky(ai + aj))
    ai_ref[0] = ai
    ai_ref[1] = jnp.concatenate([ai[:, H:], ai[:, :H]], axis=1)
    g_ref[0, :, :H] = aj[:, :H]
    g_ref[0, :, H:] = xl[:, :H]
    g_ref[1, :, :H] = aj[:, H:]
    g_ref[1, :, H:] = xl[:, H:]
    ps_ref[...] = ps


def _combine(acc_ref, ps_ref, gold_ref):
    ps = ps_ref[...]
    den = jnp.concatenate([acc_ref[0, :, :H], acc_ref[1, :, :H]], axis=1) + ps
    xl = jnp.concatenate([gold_ref[0, :, H:], gold_ref[1, :, H:]], axis=1)
    num = (jnp.concatenate([acc_ref[0, :, H:], acc_ref[1, :, H:]], axis=1)
           + ps * xl)
    return num / (den + 1e-16)


def _pre_body(t_ref, wl_ref, bl_ref, wa1_ref, wa2_ref, ba_ref,
              ai_ref, g_ref, ps_ref):
    _project(t_ref[...], wl_ref, bl_ref, wa1_ref, wa2_ref, ba_ref,
             ai_ref, g_ref, ps_ref)


_W_SPECS = [
    pl.BlockSpec((D, D), lambda i: (0, 0)),
    pl.BlockSpec((1, D), lambda i: (0, 0)),
    pl.BlockSpec((D, D), lambda i: (0, 0)),
    pl.BlockSpec((D, D), lambda i: (0, 0)),
    pl.BlockSpec((1, D), lambda i: (0, 0)),
]
_STATE_SHAPES = (
    jax.ShapeDtypeStruct((NCORES, N, D), jnp.float32),
    jax.ShapeDtypeStruct((NCORES, N, 2 * H), jnp.float32),
    jax.ShapeDtypeStruct((N, D), jnp.float32),
)
_STATE_SPECS = [
    pl.BlockSpec((NCORES, RB, D), lambda i: (0, i, 0)),
    pl.BlockSpec((NCORES, RB, 2 * H), lambda i: (0, i, 0)),
    pl.BlockSpec((RB, D), lambda i: (i, 0)),
]

_pre = pl.pallas_call(
    _pre_body,
    grid=(NBLK,),
    in_specs=[pl.BlockSpec((RB, D), lambda i: (i, 0))] + _W_SPECS,
    out_specs=_STATE_SPECS,
    out_shape=_STATE_SHAPES,
)


def _step_body(acc_ref, ps_ref, gold_ref, wl_ref, bl_ref, wa1_ref, wa2_ref,
               ba_ref, ai_ref, g_ref, ps_ref_out):
    t = _combine(acc_ref, ps_ref, gold_ref)
    _project(t, wl_ref, bl_ref, wa1_ref, wa2_ref, ba_ref,
             ai_ref, g_ref, ps_ref_out)


_step = pl.pallas_call(
    _step_body,
    grid=(NBLK,),
    in_specs=[
        pl.BlockSpec((NCORES, RB, 2 * H), lambda i: (0, i, 0)),
        pl.BlockSpec((RB, D), lambda i: (i, 0)),
        pl.BlockSpec((NCORES, RB, 2 * H), lambda i: (0, i, 0)),
    ] + _W_SPECS,
    out_specs=_STATE_SPECS,
    out_shape=_STATE_SHAPES,
)


def _edge_body(src_hbm, dst_hbm, ai_hbm, g_hbm, out_hbm,
               srcv, dstv, dsts, aib, gb, ob, acc,
               semg0, semg1, semi0, semi1, sems0, sems1):
    c = lax.axis_index("c")
    s = lax.axis_index("s")
    semg = (semg0, semg1)
    semi = (semi0, semi1)
    sems = (sems0, sems1)

    def _zrow(i, carry):
        for k in range(2 * H // 16):
            ob[0, i, pl.ds(k * 16, 16)] = jnp.zeros((16,), jnp.float32)
        return carry
    lax.fori_loop(0, CHUNK, _zrow, 0)
    nfull = ROWS_PT // CHUNK
    for j in range(nfull):
        pltpu.sync_copy(ob.at[0], acc.at[pl.ds(s * ROWS_PT + j * CHUNK, CHUNK)])
    rem = ROWS_PT - nfull * CHUNK
    if rem:
        pltpu.sync_copy(ob.at[0, pl.ds(0, rem)],
                        acc.at[pl.ds(s * ROWS_PT + nfull * CHUNK, rem)])

    plsc.subcore_barrier()

    coff = c * N

    def _issue_idx(itb, b):
        pltpu.async_copy(src_hbm.at[s, itb], srcv.at[b], semi[b])
        pltpu.async_copy(dst_hbm.at[s, itb], dstv.at[b], semi[b])

    def _wait_idx(itb, b):
        pltpu.make_async_copy(src_hbm.at[s, itb], srcv.at[b], semi[b]).wait()
        pltpu.make_async_copy(dst_hbm.at[s, itb], dstv.at[b], semi[b]).wait()

    def _shift_and_gather(b):
        for k in range(CHUNK // 16):
            srcv[b, pl.ds(k * 16, 16)] = srcv[b, pl.ds(k * 16, 16)] + coff
            dstv[b, pl.ds(k * 16, 16)] = dstv[b, pl.ds(k * 16, 16)] + coff
        pltpu.async_copy(ai_hbm.at[dstv.at[b]], aib.at[b], semg[b])
        pltpu.async_copy(g_hbm.at[srcv.at[b]], gb.at[b], semg[b])

    _issue_idx(0, 0)
    _issue_idx(1, 1)
    _wait_idx(0, 0)
    _shift_and_gather(0)

    def _outer(it, carry0):
        for b in range(2):
            itb = it * 2 + b
            nb = 1 - b

            @pl.when(itb + 1 < NCHUNK)
            def _():
                _wait_idx(itb + 1, nb)
                _shift_and_gather(nb)

            pltpu.make_async_copy(ai_hbm.at[dstv.at[b]], aib.at[b], semg[b]).wait()
            pltpu.make_async_copy(g_hbm.at[srcv.at[b]], gb.at[b], semg[b]).wait()

            @pl.when(itb >= 2)
            def _():
                pltpu.make_async_copy(ob.at[b], acc.at[dsts.at[b]],
                                      sems[b]).wait()

            for k in range(CHUNK // 16):
                dsts[b, pl.ds(k * 16, 16)] = dstv[b, pl.ds(k * 16, 16)] - coff

            @functools.partial(plsc.parallel_loop, 0, CHUNK, unroll=4)
            def _edge(e):
                for k in range(H // 16):
                    a = aib[b, e, pl.ds(k * 16, 16)]
                    v = gb[b, e, pl.ds(k * 16, 16)]
                    al = a + v
                    al = jnp.maximum(al, 0.2 * al)
                    p = jnp.exp(al)
                    xv = gb[b, e, pl.ds(H + k * 16, 16)]
                    ob[b, e, pl.ds(k * 16, 16)] = p
                    ob[b, e, pl.ds(H + k * 16, 16)] = p * xv

            pltpu.async_copy(ob.at[b], acc.at[dsts.at[b]], sems[b], add=True)

            @pl.when(itb + 2 < NCHUNK)
            def _():
                _issue_idx(itb + 2, b)
        return carry0
    lax.fori_loop(0, NCHUNK // 2, _outer, 0)

    for b in range(2):
        pltpu.make_async_copy(ob.at[b], acc.at[dsts.at[b]], sems[b]).wait()

    plsc.subcore_barrier()
    pltpu.sync_copy(acc.at[pl.ds(s * ROWS_PT, ROWS_PT)],
                    out_hbm.at[c, pl.ds(s * ROWS_PT, ROWS_PT)])


def _make_edge():
    return pl.kernel(
            _edge_body,
        out_type=jax.ShapeDtypeStruct((NCORES, N, 2 * H), jnp.float32),
        mesh=plsc.VectorSubcoreMesh(core_axis_name="c", subcore_axis_name="s"),
        compiler_params=pltpu.CompilerParams(use_tc_tiling_on_sc=False),
        scratch_types=[
            pltpu.VMEM((2, CHUNK), jnp.int32),
            pltpu.VMEM((2, CHUNK), jnp.int32),
            pltpu.VMEM((2, CHUNK), jnp.int32),
            pltpu.VMEM((2, CHUNK, D), jnp.float32),
            pltpu.VMEM((2, CHUNK, 2 * H), jnp.float32),
            pltpu.VMEM((2, CHUNK, 2 * H), jnp.float32),
            pltpu.VMEM_SHARED((N, 2 * H), jnp.float32),
            pltpu.SemaphoreType.DMA,
            pltpu.SemaphoreType.DMA,
            pltpu.SemaphoreType.DMA,
            pltpu.SemaphoreType.DMA,
            pltpu.SemaphoreType.DMA,
            pltpu.SemaphoreType.DMA,
    ],
    )


def _head_body(acc_ref, ps_ref, gold_ref, wp_ref, wn_ref, h_ref, p_ref):
    xg = _combine(acc_ref, ps_ref, gold_ref)
    h_ref[0] = jnp.mean(xg, axis=0, keepdims=True)
    xs = xg[1:, :]
    pp = jnp.dot(xs, wp_ref[...].T, preferred_element_type=jnp.float32)
    pn = jnp.dot(xs, wn_ref[...].T, preferred_element_type=jnp.float32)
    dn = (((1,), (1,)), ((), ()))
    s1 = lax.dot_general(pp, xs, dn, preferred_element_type=jnp.float32)
    s2 = lax.dot_general(pn, xs, dn, preferred_element_type=jnp.float32)
    row = lax.broadcasted_iota(jnp.int32, (NSEQ, NSEQ), 0)
    col = lax.broadcasted_iota(jnp.int32, (NSEQ, NSEQ), 1)
    eye = row == col
    ninf = jnp.float32(-jnp.inf)
    s1 = jnp.where(eye, ninf, s1)
    s2 = jnp.where(eye, ninf, s2)
    m = jnp.maximum(jnp.max(s1), jnp.max(s2))
    e1 = jnp.exp(s1 - m)
    e2 = jnp.exp(s2 - m)
    inv = 1.0 / (jnp.sum(e1) + jnp.sum(e2))
    p_ref[0, 0] = e1 * inv
    p_ref[0, 1] = e2 * inv


_head = pl.pallas_call(
    _head_body,
    grid=(NG,),
    in_specs=[
        pl.BlockSpec((NCORES, NPER, 2 * H), lambda b: (0, b, 0)),
        pl.BlockSpec((NPER, D), lambda b: (b, 0)),
        pl.BlockSpec((NCORES, NPER, 2 * H), lambda b: (0, b, 0)),
        pl.BlockSpec((D, D), lambda b: (0, 0)),
        pl.BlockSpec((D, D), lambda b: (0, 0)),
    ],
    out_specs=[
        pl.BlockSpec((1, 1, D), lambda b: (b, 0, 0)),
        pl.BlockSpec((1, 2, NSEQ, NSEQ), lambda b: (b, 0, 0, 0)),
    ],
    out_shape=(
        jax.ShapeDtypeStruct((NG, 1, D), jnp.float32),
        jax.ShapeDtypeStruct((NG, 2, NSEQ, NSEQ), jnp.float32),
    ),
)


_EDGE_CACHE = []


def _run_edge(edges, ai, g):
    if not _EDGE_CACHE:
        _EDGE_CACHE.append(_make_edge())
    return _EDGE_CACHE[0](edges[0], edges[1], ai.reshape(NCORES * N, D),
                          g.reshape(NCORES * N, 2 * H))


def kernel(x, edge_index_r0, edge_index_r1, edge_index_n,
           W_g0_lin, b_g0_lin, W_g0_attn, b_g0_attn,
           W_gn_lin, b_gn_lin, W_gn_attn, b_gn_attn,
           W_g1_lin, b_g1_lin, W_g1_attn, b_g1_attn,
           W_g2_lin, b_g2_lin, W_g2_attn, b_g2_attn,
           W_prev, W_next):
    def prep_edges(ei):
        ei = ei.astype(jnp.int32)
        return (ei[0].reshape(NSUB, NCHUNK, CHUNK),
                ei[1].reshape(NSUB, NCHUNK, CHUNK))

    e_n = prep_edges(edge_index_n)
    e_r0 = prep_edges(edge_index_r0)
    e_r1 = prep_edges(edge_index_r1)

    def prep_w(wl, bl, wa, ba):
        return (wl, bl.reshape(1, D), wa[:, :D], wa[:, D:], ba.reshape(1, D))

    w_g0 = prep_w(W_g0_lin, b_g0_lin, W_g0_attn, b_g0_attn)
    w_gn = prep_w(W_gn_lin, b_gn_lin, W_gn_attn, b_gn_attn)
    w_g1 = prep_w(W_g1_lin, b_g1_lin, W_g1_attn, b_g1_attn)
    w_g2 = prep_w(W_g2_lin, b_g2_lin, W_g2_attn, b_g2_attn)

    layers = ([(w_g0, e_n), (w_gn, e_n)] + [(w_g1, e_r1)] * 4
              + [(w_gn, e_n)] + [(w_g2, e_r0)] * 4 + [(w_gn, e_n)])

    ai, g, ps = _pre(x, *layers[0][0])
    acc = _run_edge(layers[0][1], ai, g)
    for wk, ek in layers[1:]:
        ai, g, ps = _step(acc, ps, g, *wk)
        acc = _run_edge(ek, ai, g)

    h, p4 = _head(acc, ps, g, W_prev, W_next)
    return (h.reshape(NG, D), p4.reshape(NG, 2 * NSEQ * NSEQ))


# speedup vs baseline: 1.0811x; 1.0811x over previous
import functools

import jax
import jax.numpy as jnp
from jax import lax
from jax.experimental import pallas as pl
from jax.experimental.pallas import tpu as pltpu
from jax.experimental.pallas import tpu_sc as plsc

N = 8000
D = 128
E = 128000
NPER = 1000
NSEQ = 999
NG = 8

NCORES = 2
NSUB = 16
H = D // 2
EPT = E // NSUB
CHUNK = 80
NCHUNK = EPT // CHUNK
ROWS_PT = N // NSUB


def _leaky(v):
    return jnp.maximum(v, 0.2 * v)


RB = 1000
NBLK = N // RB


def _project(t, wl_ref, bl_ref, wa1_ref, wa2_ref, ba_ref,
             ai_ref, g_ref, ps_ref):
    xl = jnp.dot(t, wl_ref[...].T, preferred_element_type=jnp.float32) + bl_ref[...]
    ai = jnp.dot(xl, wa1_ref[...].T, preferred_element_type=jnp.float32) + ba_ref[...]
    aj = jnp.dot(xl, wa2_ref[...].T, preferred_element_type=jnp.float32)
    ps = jnp.exp(_leaky(ai + aj))
    ai_ref[0] = ai
    ai_ref[1] = jnp.concatenate([ai[:, H:], ai[:, :H]], axis=1)
    g_ref[0, :, :H] = aj[:, :H]
    g_ref[0, :, H:] = xl[:, :H]
    g_ref[1, :, :H] = aj[:, H:]
    g_ref[1, :, H:] = xl[:, H:]
    ps_ref[...] = ps


def _combine(acc_ref, ps_ref, gold_ref):
    ps = ps_ref[...]
    den = jnp.concatenate([acc_ref[0, :, :H], acc_ref[1, :, :H]], axis=1) + ps
    xl = jnp.concatenate([gold_ref[0, :, H:], gold_ref[1, :, H:]], axis=1)
    num = (jnp.concatenate([acc_ref[0, :, H:], acc_ref[1, :, H:]], axis=1)
           + ps * xl)
    return num / (den + 1e-16)


def _pre_body(t_ref, wl_ref, bl_ref, wa1_ref, wa2_ref, ba_ref,
              ai_ref, g_ref, ps_ref):
    _project(t_ref[...], wl_ref, bl_ref, wa1_ref, wa2_ref, ba_ref,
             ai_ref, g_ref, ps_ref)


_W_SPECS = [
    pl.BlockSpec((D, D), lambda i: (0, 0)),
    pl.BlockSpec((1, D), lambda i: (0, 0)),
    pl.BlockSpec((D, D), lambda i: (0, 0)),
    pl.BlockSpec((D, D), lambda i: (0, 0)),
    pl.BlockSpec((1, D), lambda i: (0, 0)),
]
_STATE_SHAPES = (
    jax.ShapeDtypeStruct((NCORES, N, D), jnp.float32),
    jax.ShapeDtypeStruct((NCORES, N, 2 * H), jnp.float32),
    jax.ShapeDtypeStruct((N, D), jnp.float32),
)
_STATE_SPECS = [
    pl.BlockSpec((NCORES, RB, D), lambda i: (0, i, 0)),
    pl.BlockSpec((NCORES, RB, 2 * H), lambda i: (0, i, 0)),
    pl.BlockSpec((RB, D), lambda i: (i, 0)),
]

_pre = pl.pallas_call(
    _pre_body,
    grid=(NBLK,),
    in_specs=[pl.BlockSpec((RB, D), lambda i: (i, 0))] + _W_SPECS,
    out_specs=_STATE_SPECS,
    out_shape=_STATE_SHAPES,
)


def _step_body(acc_ref, ps_ref, gold_ref, wl_ref, bl_ref, wa1_ref, wa2_ref,
               ba_ref, ai_ref, g_ref, ps_ref_out):
    t = _combine(acc_ref, ps_ref, gold_ref)
    _project(t, wl_ref, bl_ref, wa1_ref, wa2_ref, ba_ref,
             ai_ref, g_ref, ps_ref_out)


_step = pl.pallas_call(
    _step_body,
    grid=(NBLK,),
    in_specs=[
        pl.BlockSpec((NCORES, RB, 2 * H), lambda i: (0, i, 0)),
        pl.BlockSpec((RB, D), lambda i: (i, 0)),
        pl.BlockSpec((NCORES, RB, 2 * H), lambda i: (0, i, 0)),
    ] + _W_SPECS,
    out_specs=_STATE_SPECS,
    out_shape=_STATE_SHAPES,
)


def _edge_body(src_hbm, dst_hbm, ai_hbm, g_hbm, out_hbm,
               srcv, dstv, dstg, dsts, aib, gb, ob, acc,
               semg0, semg1, semi0, semi1, sems0, sems1):
    c = lax.axis_index("c")
    s = lax.axis_index("s")
    semg = (semg0, semg1)
    semi = (semi0, semi1)
    sems = (sems0, sems1)

    def _zrow(i, carry):
        for k in range(2 * H // 16):
            ob[0, i, pl.ds(k * 16, 16)] = jnp.zeros((16,), jnp.float32)
        return carry
    lax.fori_loop(0, CHUNK, _zrow, 0)
    nfull = ROWS_PT // CHUNK
    for j in range(nfull):
        pltpu.sync_copy(ob.at[0], acc.at[pl.ds(s * ROWS_PT + j * CHUNK, CHUNK)])
    rem = ROWS_PT - nfull * CHUNK
    if rem:
        pltpu.sync_copy(ob.at[0, pl.ds(0, rem)],
                        acc.at[pl.ds(s * ROWS_PT + nfull * CHUNK, rem)])

    plsc.subcore_barrier()

    coff = c * N

    def _issue_idx(itb, b):
        pltpu.async_copy(src_hbm.at[s, itb], srcv.at[b], semi[b])
        pltpu.async_copy(dst_hbm.at[s, itb], dstv.at[b], semi[b])

    def _wait_idx(itb, b):
        pltpu.make_async_copy(src_hbm.at[s, itb], srcv.at[b], semi[b]).wait()
        pltpu.make_async_copy(dst_hbm.at[s, itb], dstv.at[b], semi[b]).wait()

    def _shift_and_gather(b):
        for k in range(CHUNK // 16):
            sl = pl.ds(k * 16, 16)
            srcv[b, sl] = srcv[b, sl] + coff
            dstg[b, sl] = (dstv[b, sl] + coff) * 2
        pltpu.async_copy(ai_hbm.at[dstg.at[b]], aib.at[b], semg[b])
        pltpu.async_copy(g_hbm.at[srcv.at[b]], gb.at[b], semg[b])

    _issue_idx(0, 0)
    _issue_idx(1, 1)
    _wait_idx(0, 0)
    _shift_and_gather(0)

    def _outer(it, carry0):
        for b in range(2):
            itb = it * 2 + b
            nb = 1 - b

            @pl.when(itb + 1 < NCHUNK)
            def _():
                _wait_idx(itb + 1, nb)
                _shift_and_gather(nb)

            pltpu.make_async_copy(ai_hbm.at[dstg.at[b]], aib.at[b], semg[b]).wait()
            pltpu.make_async_copy(g_hbm.at[srcv.at[b]], gb.at[b], semg[b]).wait()

            @pl.when(itb >= 2)
            def _():
                pltpu.make_async_copy(ob.at[b], acc.at[dsts.at[b]],
                                      sems[b]).wait()

            for k in range(CHUNK // 16):
                dsts[b, pl.ds(k * 16, 16)] = dstv[b, pl.ds(k * 16, 16)]

            @functools.partial(plsc.parallel_loop, 0, CHUNK, unroll=4)
            def _edge(e):
                for k in range(H // 16):
                    a = aib[b, e, pl.ds(k * 16, 16)]
                    v = gb[b, e, pl.ds(k * 16, 16)]
                    al = a + v
                    al = jnp.maximum(al, 0.2 * al)
                    p = jnp.exp(al)
                    xv = gb[b, e, pl.ds(H + k * 16, 16)]
                    ob[b, e, pl.ds(k * 16, 16)] = p
                    ob[b, e, pl.ds(H + k * 16, 16)] = p * xv

            pltpu.async_copy(ob.at[b], acc.at[dsts.at[b]], sems[b], add=True)

            @pl.when(itb + 2 < NCHUNK)
            def _():
                _issue_idx(itb + 2, b)
        return carry0
    lax.fori_loop(0, NCHUNK // 2, _outer, 0)

    for b in range(2):
        pltpu.make_async_copy(ob.at[b], acc.at[dsts.at[b]], sems[b]).wait()

    plsc.subcore_barrier()
    pltpu.sync_copy(acc.at[pl.ds(s * ROWS_PT, ROWS_PT)],
                    out_hbm.at[c, pl.ds(s * ROWS_PT, ROWS_PT)])


def _make_edge():
    return pl.kernel(
            _edge_body,
        out_type=jax.ShapeDtypeStruct((NCORES, N, 2 * H), jnp.float32),
        mesh=plsc.VectorSubcoreMesh(core_axis_name="c", subcore_axis_name="s"),
        compiler_params=pltpu.CompilerParams(use_tc_tiling_on_sc=False),
        scratch_types=[
            pltpu.VMEM((2, CHUNK), jnp.int32),
            pltpu.VMEM((2, CHUNK), jnp.int32),
            pltpu.VMEM((2, CHUNK), jnp.int32),
            pltpu.VMEM((2, CHUNK), jnp.int32),
            pltpu.VMEM((2, CHUNK, H), jnp.float32),
            pltpu.VMEM((2, CHUNK, 2 * H), jnp.float32),
            pltpu.VMEM((2, CHUNK, 2 * H), jnp.float32),
            pltpu.VMEM_SHARED((N, 2 * H), jnp.float32),
            pltpu.SemaphoreType.DMA,
            pltpu.SemaphoreType.DMA,
            pltpu.SemaphoreType.DMA,
            pltpu.SemaphoreType.DMA,
            pltpu.SemaphoreType.DMA,
            pltpu.SemaphoreType.DMA,
    ],
    )


def _head_body(acc_ref, ps_ref, gold_ref, wp_ref, wn_ref, h_ref, p_ref):
    xg = _combine(acc_ref, ps_ref, gold_ref)
    h_ref[0] = jnp.mean(xg, axis=0, keepdims=True)
    xs = xg[1:, :]
    pp = jnp.dot(xs, wp_ref[...].T, preferred_element_type=jnp.float32)
    pn = jnp.dot(xs, wn_ref[...].T, preferred_element_type=jnp.float32)
    dn = (((1,), (1,)), ((), ()))
    s1 = lax.dot_general(pp, xs, dn, preferred_element_type=jnp.float32)
    s2 = lax.dot_general(pn, xs, dn, preferred_element_type=jnp.float32)
    row = lax.broadcasted_iota(jnp.int32, (NSEQ, NSEQ), 0)
    col = lax.broadcasted_iota(jnp.int32, (NSEQ, NSEQ), 1)
    eye = row == col
    ninf = jnp.float32(-jnp.inf)
    s1 = jnp.where(eye, ninf, s1)
    s2 = jnp.where(eye, ninf, s2)
    m = jnp.maximum(jnp.max(s1), jnp.max(s2))
    e1 = jnp.exp(s1 - m)
    e2 = jnp.exp(s2 - m)
    inv = 1.0 / (jnp.sum(e1) + jnp.sum(e2))
    p_ref[0, 0] = e1 * inv
    p_ref[0, 1] = e2 * inv


_head = pl.pallas_call(
    _head_body,
    grid=(NG,),
    in_specs=[
        pl.BlockSpec((NCORES, NPER, 2 * H), lambda b: (0, b, 0)),
        pl.BlockSpec((NPER, D), lambda b: (b, 0)),
        pl.BlockSpec((NCORES, NPER, 2 * H), lambda b: (0, b, 0)),
        pl.BlockSpec((D, D), lambda b: (0, 0)),
        pl.BlockSpec((D, D), lambda b: (0, 0)),
    ],
    out_specs=[
        pl.BlockSpec((1, 1, D), lambda b: (b, 0, 0)),
        pl.BlockSpec((1, 2, NSEQ, NSEQ), lambda b: (b, 0, 0, 0)),
    ],
    out_shape=(
        jax.ShapeDtypeStruct((NG, 1, D), jnp.float32),
        jax.ShapeDtypeStruct((NG, 2, NSEQ, NSEQ), jnp.float32),
    ),
)


_EDGE_CACHE = []


def _run_edge(edges, ai, g):
    if not _EDGE_CACHE:
        _EDGE_CACHE.append(_make_edge())
    return _EDGE_CACHE[0](edges[0], edges[1], ai.reshape(2 * NCORES * N, H),
                          g.reshape(NCORES * N, 2 * H))


def kernel(x, edge_index_r0, edge_index_r1, edge_index_n,
           W_g0_lin, b_g0_lin, W_g0_attn, b_g0_attn,
           W_gn_lin, b_gn_lin, W_gn_attn, b_gn_attn,
           W_g1_lin, b_g1_lin, W_g1_attn, b_g1_attn,
           W_g2_lin, b_g2_lin, W_g2_attn, b_g2_attn,
           W_prev, W_next):
    def prep_edges(ei):
        ei = ei.astype(jnp.int32)
        return (ei[0].reshape(NSUB, NCHUNK, CHUNK),
                ei[1].reshape(NSUB, NCHUNK, CHUNK))

    e_n = prep_edges(edge_index_n)
    e_r0 = prep_edges(edge_index_r0)
    e_r1 = prep_edges(edge_index_r1)

    def prep_w(wl, bl, wa, ba):
        return (wl, bl.reshape(1, D), wa[:, :D], wa[:, D:], ba.reshape(1, D))

    w_g0 = prep_w(W_g0_lin, b_g0_lin, W_g0_attn, b_g0_attn)
    w_gn = prep_w(W_gn_lin, b_gn_lin, W_gn_attn, b_gn_attn)
    w_g1 = prep_w(W_g1_lin, b_g1_lin, W_g1_attn, b_g1_attn)
    w_g2 = prep_w(W_g2_lin, b_g2_lin, W_g2_attn, b_g2_attn)

    layers = ([(w_g0, e_n), (w_gn, e_n)] + [(w_g1, e_r1)] * 4
              + [(w_gn, e_n)] + [(w_g2, e_r0)] * 4 + [(w_gn, e_n)])

    ai, g, ps = _pre(x, *layers[0][0])
    acc = _run_edge(layers[0][1], ai, g)
    for wk, ek in layers[1:]:
        ai, g, ps = _step(acc, ps, g, *wk)
        acc = _run_edge(ek, ai, g)

    h, p4 = _head(acc, ps, g, W_prev, W_next)
    return (h.reshape(NG, D), p4.reshape(NG, 2 * NSEQ * NSEQ))
